# baseline (speedup 1.0000x reference)
import jax
import jax.numpy as jnp
from jax.experimental import pallas as pl
from jax.experimental.pallas import tpu as pltpu

N = 4096
D = 64
TAU = 0.5
BLK = 512
NB = N // BLK
LOG2E = 1.4426950408889634
SCALE = LOG2E / TAU


def _prep_body(z1_ref, z2_ref, w1_ref, b1_ref, w2_ref, b2_ref,
               z1pn_ref, z2pn_ref, t_ref):
    w1t = w1_ref[...].T
    w2t = w2_ref[...].T
    b1 = b1_ref[...]
    b2 = b2_ref[...]

    def proj_norm(z, scale):
        h = jnp.maximum(
            jax.lax.dot(z, w1t, preferred_element_type=jnp.float32) + b1, 0.0)
        zp = jax.lax.dot(h, w2t, preferred_element_type=jnp.float32) + b2
        rn = jax.lax.rsqrt(jnp.sum(zp * zp, axis=1, keepdims=True)) * scale
        return zp * rn

    z1pn = proj_norm(z1_ref[...], SCALE)
    z2pn = proj_norm(z2_ref[...], 1.0)
    z1pn_ref[...] = z1pn.astype(jnp.bfloat16)
    z2pn_ref[...] = z2pn.astype(jnp.bfloat16)
    s1 = jnp.sum(z1pn, axis=0, keepdims=True)
    s2 = jnp.sum(z2pn, axis=0, keepdims=True)
    t_ref[0] = jnp.sum(s1 * s2)


def _main_body(z1pn_ref, z2pn_ref, t_ref, out_ref, acc_ref):
    i = pl.program_id(0)

    @pl.when(i == 0)
    def _init():
        acc_ref[0] = 0.0

    s2 = jax.lax.dot_general(z1pn_ref[...], z2pn_ref[...],
                             (((1,), (1,)), ((), ())),
                             preferred_element_type=jnp.float32)
    rowsum = jnp.sum(jnp.exp2(s2), axis=1, keepdims=True)
    acc_ref[0] += jnp.sum(jnp.log(rowsum + 1e-8))

    @pl.when(i == NB - 1)
    def _fin():
        out_ref[0] = (-t_ref[0] / LOG2E) / (N * N) + acc_ref[0] / N


@jax.jit
def kernel(z_mp_i1, z_mp_i2, W1, b1, W2, b2):
    b1r = b1.reshape(1, D)
    b2r = b2.reshape(1, D)
    z1pn, z2pn, t = pl.pallas_call(
        _prep_body,
        out_shape=(
            jax.ShapeDtypeStruct((N, D), jnp.bfloat16),
            jax.ShapeDtypeStruct((N, D), jnp.bfloat16),
            jax.ShapeDtypeStruct((1,), jnp.float32),
        ),
        out_specs=(
            pl.BlockSpec((N, D), lambda: (0, 0)),
            pl.BlockSpec((N, D), lambda: (0, 0)),
            pl.BlockSpec(memory_space=pltpu.SMEM),
        ),
    )(z_mp_i1, z_mp_i2, W1, b1r, W2, b2r)

    out = pl.pallas_call(
        _main_body,
        grid=(NB,),
        in_specs=[
            pl.BlockSpec((BLK, D), lambda i: (i, 0)),
            pl.BlockSpec((N, D), lambda i: (0, 0)),
            pl.BlockSpec(memory_space=pltpu.SMEM),
        ],
        out_specs=pl.BlockSpec(memory_space=pltpu.SMEM),
        out_shape=jax.ShapeDtypeStruct((1,), jnp.float32),
        scratch_shapes=[
            pltpu.SMEM((1,), jnp.float32),
        ],
    )(z1pn, z2pn, t)
    return out[0]


# speedup vs baseline: 1.0765x; 1.0765x over previous
import jax
import jax.numpy as jnp
from jax.experimental import pallas as pl
from jax.experimental.pallas import tpu as pltpu

N = 4096
D = 64
TAU = 0.5
BLK = 512
NB = N // BLK
LOG2E = 1.4426950408889634
SCALE = LOG2E / TAU


def _ssl_body(z1_ref, z2_ref, w1_ref, b1_ref, w2_ref, b2_ref,
              out_ref, z2pn_ref, z2sum_ref, z1sum_ref, acc_ref):
    i = pl.program_id(0)

    w1t = w1_ref[...].T
    w2t = w2_ref[...].T
    b1 = b1_ref[...]
    b2 = b2_ref[...]

    def proj_norm(z, scale):
        h = jnp.maximum(
            jax.lax.dot(z, w1t, preferred_element_type=jnp.float32) + b1, 0.0)
        zp = jax.lax.dot(h, w2t, preferred_element_type=jnp.float32) + b2
        rn = jax.lax.rsqrt(jnp.sum(zp * zp, axis=1, keepdims=True)) * scale
        return zp * rn

    @pl.when(i == 0)
    def _init():
        z2pn = proj_norm(z2_ref[...], 1.0)
        z2pn_ref[...] = z2pn.astype(jnp.bfloat16)
        z2sum_ref[...] = jnp.sum(z2pn, axis=0, keepdims=True)
        z1sum_ref[...] = jnp.zeros_like(z1sum_ref)
        acc_ref[0] = 0.0

    z1pn = proj_norm(z1_ref[...], SCALE)
    z1sum_ref[...] += jnp.sum(z1pn, axis=0, keepdims=True)

    s2 = jax.lax.dot_general(z1pn.astype(jnp.bfloat16), z2pn_ref[...],
                             (((1,), (1,)), ((), ())),
                             preferred_element_type=jnp.float32)
    rowsum = jnp.sum(jnp.exp2(s2), axis=1, keepdims=True)
    acc_ref[0] += jnp.sum(jnp.log(rowsum + 1e-8))

    @pl.when(i == NB - 1)
    def _fin():
        total_s = jnp.sum(z1sum_ref[...] * z2sum_ref[...]) * (1.0 / LOG2E)
        out_ref[0] = -total_s / (N * N) + acc_ref[0] / N


@jax.jit
def kernel(z_mp_i1, z_mp_i2, W1, b1, W2, b2):
    b1r = b1.reshape(1, D)
    b2r = b2.reshape(1, D)
    out = pl.pallas_call(
        _ssl_body,
        grid=(NB,),
        in_specs=[
            pl.BlockSpec((BLK, D), lambda i: (i, 0)),
            pl.BlockSpec((N, D), lambda i: (0, 0)),
            pl.BlockSpec((D, D), lambda i: (0, 0)),
            pl.BlockSpec((1, D), lambda i: (0, 0)),
            pl.BlockSpec((D, D), lambda i: (0, 0)),
            pl.BlockSpec((1, D), lambda i: (0, 0)),
        ],
        out_specs=pl.BlockSpec(memory_space=pltpu.SMEM),
        out_shape=jax.ShapeDtypeStruct((1,), jnp.float32),
        scratch_shapes=[
            pltpu.VMEM((N, D), jnp.bfloat16),
            pltpu.VMEM((1, D), jnp.float32),
            pltpu.VMEM((1, D), jnp.float32),
            pltpu.SMEM((2,), jnp.float32),
        ],
    )(z_mp_i1, z_mp_i2, W1, b1r, W2, b2r)
    return out[0]


# speedup vs baseline: 1.1340x; 1.0534x over previous
import jax
import jax.numpy as jnp
from jax.experimental import pallas as pl
from jax.experimental.pallas import tpu as pltpu

N = 4096
D = 64
TAU = 0.5
BLK = 512
NB = N // BLK
LOG2E = 1.4426950408889634
SCALE = LOG2E / TAU


def _ssl_body(z1_ref, z2_ref, w1_ref, b1_ref, w2_ref, b2_ref,
              out_ref, z1pn_ref, z2pn_ref, acc_ref):
    i = pl.program_id(0)

    @pl.when(i == 0)
    def _init():
        w1t = w1_ref[...].T
        w2t = w2_ref[...].T
        b1 = b1_ref[...]
        b2 = b2_ref[...]

        def proj_norm(z, scale):
            h = jnp.maximum(
                jax.lax.dot(z, w1t, preferred_element_type=jnp.float32) + b1,
                0.0)
            zp = jax.lax.dot(h, w2t, preferred_element_type=jnp.float32) + b2
            rn = jax.lax.rsqrt(jnp.sum(zp * zp, axis=1, keepdims=True)) * scale
            return zp * rn

        z1pn = proj_norm(z1_ref[...], SCALE)
        z2pn = proj_norm(z2_ref[...], 1.0)
        z1pn_ref[...] = z1pn.astype(jnp.bfloat16)
        z2pn_ref[...] = z2pn.astype(jnp.bfloat16)
        s1 = jnp.sum(z1pn, axis=0, keepdims=True)
        s2 = jnp.sum(z2pn, axis=0, keepdims=True)
        acc_ref[1] = jnp.sum(s1 * s2) * (1.0 / LOG2E)
        acc_ref[0] = 0.0

    sblk = jax.lax.dot_general(z1pn_ref[pl.ds(i * BLK, BLK), :], z2pn_ref[...],
                               (((1,), (1,)), ((), ())),
                               preferred_element_type=jnp.float32)
    rowsum = jnp.sum(jnp.exp2(sblk), axis=1, keepdims=True)
    acc_ref[0] += jnp.sum(jnp.log(rowsum + 1e-8))

    @pl.when(i == NB - 1)
    def _fin():
        out_ref[0] = -acc_ref[1] / (N * N) + acc_ref[0] / N


@jax.jit
def kernel(z_mp_i1, z_mp_i2, W1, b1, W2, b2):
    b1r = b1.reshape(1, D)
    b2r = b2.reshape(1, D)
    out = pl.pallas_call(
        _ssl_body,
        grid=(NB,),
        in_specs=[
            pl.BlockSpec((N, D), lambda i: (0, 0)),
            pl.BlockSpec((N, D), lambda i: (0, 0)),
            pl.BlockSpec((D, D), lambda i: (0, 0)),
            pl.BlockSpec((1, D), lambda i: (0, 0)),
            pl.BlockSpec((D, D), lambda i: (0, 0)),
            pl.BlockSpec((1, D), lambda i: (0, 0)),
        ],
        out_specs=pl.BlockSpec(memory_space=pltpu.SMEM),
        out_shape=jax.ShapeDtypeStruct((1,), jnp.float32),
        scratch_shapes=[
            pltpu.VMEM((N, D), jnp.bfloat16),
            pltpu.VMEM((N, D), jnp.bfloat16),
            pltpu.SMEM((2,), jnp.float32),
        ],
    )(z_mp_i1, z_mp_i2, W1, b1r, W2, b2r)
    return out[0]


# speedup vs baseline: 1.2164x; 1.0727x over previous
import jax
import jax.numpy as jnp
from jax.experimental import pallas as pl
from jax.experimental.pallas import tpu as pltpu

N = 4096
D = 64
TAU = 0.5
BLK = 1024
NB = N // BLK
LOG2E = 1.4426950408889634
SCALE = LOG2E / TAU


def _ssl_body(z1_ref, z2_ref, w1_ref, b1_ref, w2_ref, b2_ref,
              out_ref, z1pn_ref, z2pn_ref, acc_ref):
    i = pl.program_id(0)

    @pl.when(i == 0)
    def _init():
        w1t = w1_ref[...].T
        w2t = w2_ref[...].T
        b1 = b1_ref[...]
        b2 = b2_ref[...]

        def proj_norm(z, scale):
            h = jnp.maximum(
                jax.lax.dot(z, w1t, preferred_element_type=jnp.float32) + b1,
                0.0)
            zp = jax.lax.dot(h, w2t, preferred_element_type=jnp.float32) + b2
            rn = jax.lax.rsqrt(jnp.sum(zp * zp, axis=1, keepdims=True)) * scale
            return zp * rn

        z1pn = proj_norm(z1_ref[...], SCALE)
        z2pn = proj_norm(z2_ref[...], 1.0)
        z1pn_ref[...] = z1pn.astype(jnp.bfloat16)
        z2pn_ref[...] = z2pn.astype(jnp.bfloat16)
        s1 = jnp.sum(z1pn, axis=0, keepdims=True)
        s2 = jnp.sum(z2pn, axis=0, keepdims=True)
        acc_ref[1] = jnp.sum(s1 * s2) * (1.0 / LOG2E)
        acc_ref[0] = 0.0

    sblk = jax.lax.dot_general(z1pn_ref[pl.ds(i * BLK, BLK), :], z2pn_ref[...],
                               (((1,), (1,)), ((), ())),
                               preferred_element_type=jnp.float32)
    rowsum = jnp.sum(jnp.exp2(sblk), axis=1, keepdims=True)
    acc_ref[0] += jnp.sum(jnp.log(rowsum + 1e-8))

    @pl.when(i == NB - 1)
    def _fin():
        out_ref[0] = -acc_ref[1] / (N * N) + acc_ref[0] / N


@jax.jit
def kernel(z_mp_i1, z_mp_i2, W1, b1, W2, b2):
    b1r = b1.reshape(1, D)
    b2r = b2.reshape(1, D)
    out = pl.pallas_call(
        _ssl_body,
        grid=(NB,),
        in_specs=[
            pl.BlockSpec((N, D), lambda i: (0, 0)),
            pl.BlockSpec((N, D), lambda i: (0, 0)),
            pl.BlockSpec((D, D), lambda i: (0, 0)),
            pl.BlockSpec((1, D), lambda i: (0, 0)),
            pl.BlockSpec((D, D), lambda i: (0, 0)),
            pl.BlockSpec((1, D), lambda i: (0, 0)),
        ],
        out_specs=pl.BlockSpec(memory_space=pltpu.SMEM),
        out_shape=jax.ShapeDtypeStruct((1,), jnp.float32),
        scratch_shapes=[
            pltpu.VMEM((N, D), jnp.bfloat16),
            pltpu.VMEM((N, D), jnp.bfloat16),
            pltpu.SMEM((2,), jnp.float32),
        ],
    )(z_mp_i1, z_mp_i2, W1, b1r, W2, b2r)
    return out[0]


# speedup vs baseline: 1.2661x; 1.0408x over previous
import jax
import jax.numpy as jnp
from jax.experimental import pallas as pl
from jax.experimental.pallas import tpu as pltpu

N = 4096
D = 64
TAU = 0.5
BLK = 2048
NB = N // BLK
LOG2E = 1.4426950408889634
SCALE = LOG2E / TAU


def _ssl_body(z1_ref, z2_ref, w1_ref, b1_ref, w2_ref, b2_ref,
              out_ref, z1pn_ref, z2pn_ref, acc_ref):
    i = pl.program_id(0)

    @pl.when(i == 0)
    def _init():
        w1t = w1_ref[...].T
        w2t = w2_ref[...].T
        b1 = b1_ref[...]
        b2 = b2_ref[...]

        def proj_norm(z, scale):
            h = jnp.maximum(
                jax.lax.dot(z, w1t, preferred_element_type=jnp.float32) + b1,
                0.0)
            zp = jax.lax.dot(h, w2t, preferred_element_type=jnp.float32) + b2
            rn = jax.lax.rsqrt(jnp.sum(zp * zp, axis=1, keepdims=True)) * scale
            return zp * rn

        z1pn = proj_norm(z1_ref[...], SCALE)
        z2pn = proj_norm(z2_ref[...], 1.0)
        z1pn_ref[...] = z1pn.astype(jnp.bfloat16)
        z2pn_ref[...] = z2pn.astype(jnp.bfloat16)
        s1 = jnp.sum(z1pn, axis=0, keepdims=True)
        s2 = jnp.sum(z2pn, axis=0, keepdims=True)
        acc_ref[1] = jnp.sum(s1 * s2) * (1.0 / LOG2E)
        acc_ref[0] = 0.0

    sblk = jax.lax.dot_general(z1pn_ref[pl.ds(i * BLK, BLK), :], z2pn_ref[...],
                               (((1,), (1,)), ((), ())),
                               preferred_element_type=jnp.float32)
    rowsum = jnp.sum(jnp.exp2(sblk), axis=1, keepdims=True)
    acc_ref[0] += jnp.sum(jnp.log(rowsum + 1e-8))

    @pl.when(i == NB - 1)
    def _fin():
        out_ref[0] = -acc_ref[1] / (N * N) + acc_ref[0] / N


@jax.jit
def kernel(z_mp_i1, z_mp_i2, W1, b1, W2, b2):
    b1r = b1.reshape(1, D)
    b2r = b2.reshape(1, D)
    out = pl.pallas_call(
        _ssl_body,
        grid=(NB,),
        in_specs=[
            pl.BlockSpec((N, D), lambda i: (0, 0)),
            pl.BlockSpec((N, D), lambda i: (0, 0)),
            pl.BlockSpec((D, D), lambda i: (0, 0)),
            pl.BlockSpec((1, D), lambda i: (0, 0)),
            pl.BlockSpec((D, D), lambda i: (0, 0)),
            pl.BlockSpec((1, D), lambda i: (0, 0)),
        ],
        out_specs=pl.BlockSpec(memory_space=pltpu.SMEM),
        out_shape=jax.ShapeDtypeStruct((1,), jnp.float32),
        scratch_shapes=[
            pltpu.VMEM((N, D), jnp.bfloat16),
            pltpu.VMEM((N, D), jnp.bfloat16),
            pltpu.SMEM((2,), jnp.float32),
        ],
    )(z_mp_i1, z_mp_i2, W1, b1r, W2, b2r)
    return out[0]
